# Initial kernel scaffold; baseline (speedup 1.0000x reference)
import jax
import jax.numpy as jnp
from jax.experimental import pallas as pl


def kernel(x, edge_index, W1, b1, W2, b2):
    raise NotImplementedError("write your pallas kernel here")


# speedup vs baseline: 21.9325x; 21.9325x over previous
import functools

import jax
import jax.numpy as jnp
from jax import lax
from jax.experimental import pallas as pl
from jax.experimental.pallas import tpu as pltpu
from jax.experimental.pallas import tpu_sc as plsc

N = 10000
D = 128
E = 320000
CH = 128
NC = 2
NS = 16
NW = NC * NS
NCH = 80
E_PAD = NW * NCH * CH
NROW = E_PAD // CH
N_ACC = 10240
RPW = N_ACC // NS
OPW = N // NS

_mesh = plsc.VectorSubcoreMesh(core_axis_name="c", subcore_axis_name="s")


@functools.partial(
    pl.kernel,
    out_type=jax.ShapeDtypeStruct((N_ACC,), jnp.float32),
    mesh=_mesh,
    compiler_params=pltpu.CompilerParams(needs_layout_passes=False),
    scratch_types=[
        pltpu.VMEM((NROW // NS, CH), jnp.int32),
        pltpu.VMEM((CH,), jnp.float32),
        pltpu.VMEM((RPW,), jnp.float32),
        pltpu.VMEM((RPW,), jnp.float32),
        pltpu.VMEM_SHARED((N_ACC,), jnp.float32),
        pltpu.SemaphoreType.DMA,
    ],
)
def _deg_dinv(dst_hbm, ones_hbm, zeros_hbm, dinv_hbm,
              dstv, onesv, degv, dinvv, deg_sh, sem):
    cid = lax.axis_index("c")
    sid = lax.axis_index("s")
    rows_per = NROW // NS

    @pl.when(cid == 0)
    def _():
        pltpu.sync_copy(ones_hbm, onesv)
        pltpu.sync_copy(dst_hbm.at[pl.ds(sid * rows_per, rows_per)], dstv)
        pltpu.sync_copy(zeros_hbm, deg_sh.at[pl.ds(sid * RPW, RPW)])
        plsc.subcore_barrier()

        def body(j, carry):
            pltpu.sync_copy(onesv, deg_sh.at[dstv.at[j]], add=True)
            return carry

        lax.fori_loop(0, rows_per, body, 0)
        plsc.subcore_barrier()

        pltpu.sync_copy(deg_sh.at[pl.ds(sid * RPW, RPW)], degv)

        def newton(k, carry):
            x = degv[pl.ds(k * 16, 16)] + 1.0
            i = plsc.bitcast(x, jnp.int32)
            y = plsc.bitcast(jnp.int32(0x5F3759DF) - (i >> 1), jnp.float32)
            for _ in range(3):
                y = y * (1.5 - 0.5 * x * y * y)
            dinvv[pl.ds(k * 16, 16)] = y
            return carry

        lax.fori_loop(0, RPW // 16, newton, 0)
        pltpu.sync_copy(dinvv, dinv_hbm.at[pl.ds(sid * RPW, RPW)])


@functools.partial(
    pl.kernel,
    out_type=jax.ShapeDtypeStruct((NC, N_ACC, D), jnp.float32),
    mesh=_mesh,
    compiler_params=pltpu.CompilerParams(needs_layout_passes=False),
    scratch_types=[
        pltpu.VMEM((NCH, CH), jnp.int32),
        pltpu.VMEM((NCH, CH), jnp.int32),
        pltpu.VMEM((CH, D), jnp.float32),
        pltpu.VMEM_SHARED((N_ACC, D), jnp.float32),
        pltpu.SemaphoreType.DMA,
    ],
)
def _segsum(g_hbm, src_hbm, dst_hbm, zeros_hbm, acc_hbm,
            srcv, dstv, rows, acc_sh, sem):
    cid = lax.axis_index("c")
    sid = lax.axis_index("s")
    wid = cid * NS + sid

    pltpu.sync_copy(zeros_hbm, acc_sh.at[pl.ds(sid * RPW, RPW)])
    pltpu.sync_copy(src_hbm.at[pl.ds(wid * NCH, NCH)], srcv)
    pltpu.sync_copy(dst_hbm.at[pl.ds(wid * NCH, NCH)], dstv)
    plsc.subcore_barrier()

    def body(j, carry):
        pltpu.async_copy(g_hbm.at[srcv.at[j]], rows, sem).wait()
        pltpu.sync_copy(rows, acc_sh.at[dstv.at[j]], add=True)
        return carry

    lax.fori_loop(0, NCH, body, 0)
    plsc.subcore_barrier()
    pltpu.sync_copy(acc_sh.at[pl.ds(sid * RPW, RPW)],
                    acc_hbm.at[cid, pl.ds(sid * RPW, RPW)])


def _scale_mm_body(x_ref, w_ref, dinv_ref, g_ref):
    g_ref[...] = jnp.dot(x_ref[...], w_ref[...],
                         preferred_element_type=jnp.float32) * dinv_ref[...]


def _mid_body(acc_ref, g_ref, dinv_ref, b_ref, w_ref, out_ref):
    o = (acc_ref[0] + acc_ref[1] + g_ref[...]) * dinv_ref[...] + b_ref[...]
    h = jnp.maximum(o, 0.0)
    out_ref[...] = jnp.dot(h, w_ref[...],
                           preferred_element_type=jnp.float32) * dinv_ref[...]


def _final_body(acc_ref, g_ref, dinv_ref, b_ref, out_ref):
    out_ref[...] = ((acc_ref[0] + acc_ref[1] + g_ref[...]) * dinv_ref[...]
                    + b_ref[...])


_BLK = 1000
_GRID = (N // _BLK,)
_row_spec = pl.BlockSpec((_BLK, D), lambda i: (i, 0))
_dinv_spec = pl.BlockSpec((_BLK, 1), lambda i: (i, 0))
_w_spec = pl.BlockSpec((D, D), lambda i: (0, 0))
_b_spec = pl.BlockSpec((1, D), lambda i: (0, 0))
_acc_spec = pl.BlockSpec((NC, _BLK, D), lambda i: (0, i, 0))
_row_out = jax.ShapeDtypeStruct((N, D), jnp.float32)

_scale_mm = pl.pallas_call(
    _scale_mm_body, grid=_GRID,
    in_specs=[_row_spec, _w_spec, _dinv_spec],
    out_specs=_row_spec, out_shape=_row_out)

_mid = pl.pallas_call(
    _mid_body, grid=_GRID,
    in_specs=[_acc_spec, _row_spec, _dinv_spec, _b_spec, _w_spec],
    out_specs=_row_spec, out_shape=_row_out)

_final = pl.pallas_call(
    _final_body, grid=_GRID,
    in_specs=[_acc_spec, _row_spec, _dinv_spec, _b_spec],
    out_specs=_row_spec, out_shape=_row_out)


def kernel(x, edge_index, W1, b1, W2, b2):
    src = edge_index[0].astype(jnp.int32)
    dst = edge_index[1].astype(jnp.int32)
    npad = E_PAD - E
    pi = jnp.arange(npad, dtype=jnp.int32)
    src2d = jnp.concatenate([src, pi % N]).reshape(NROW, CH)
    dst2d = jnp.concatenate([dst, N + pi % (N_ACC - N)]).reshape(NROW, CH)

    ones1 = jnp.ones((CH,), jnp.float32)
    zeros1 = jnp.zeros((RPW,), jnp.float32)
    zrows = jnp.zeros((RPW, D), jnp.float32)

    dinv_flat = _deg_dinv(dst2d, ones1, zeros1)
    dinv = dinv_flat[:N, None]

    g1 = _scale_mm(x, W1, dinv)
    acc1 = _segsum(g1, src2d, dst2d, zrows)
    g2 = _mid(acc1, g1, dinv, b1[None, :], W2)
    acc2 = _segsum(g2, src2d, dst2d, zrows)
    return _final(acc2, g2, dinv, b2[None, :])


# speedup vs baseline: 26.5856x; 1.2122x over previous
import functools

import jax
import jax.numpy as jnp
from jax import lax
from jax.experimental import pallas as pl
from jax.experimental.pallas import tpu as pltpu
from jax.experimental.pallas import tpu_sc as plsc

N = 10000
D = 128
E = 320000
CH = 128
NC = 2
NS = 16
NW = NC * NS
NCH = 80
E_PAD = NW * NCH * CH
NROW = E_PAD // CH
N_ACC = 10240
RPW = N_ACC // NS
OPW = N // NS

_mesh = plsc.VectorSubcoreMesh(core_axis_name="c", subcore_axis_name="s")


@functools.partial(
    pl.kernel,
    out_type=jax.ShapeDtypeStruct((N_ACC,), jnp.float32),
    mesh=_mesh,
    compiler_params=pltpu.CompilerParams(needs_layout_passes=False),
    scratch_types=[
        pltpu.VMEM((NROW // NS, CH), jnp.int32),
        pltpu.VMEM((CH,), jnp.float32),
        pltpu.VMEM((RPW,), jnp.float32),
        pltpu.VMEM((RPW,), jnp.float32),
        pltpu.VMEM_SHARED((N_ACC,), jnp.float32),
        pltpu.SemaphoreType.DMA,
    ],
)
def _deg_dinv(dst_hbm, ones_hbm, zeros_hbm, dinv_hbm,
              dstv, onesv, degv, dinvv, deg_sh, sem):
    cid = lax.axis_index("c")
    sid = lax.axis_index("s")
    rows_per = NROW // NS

    @pl.when(cid == 0)
    def _():
        pltpu.sync_copy(ones_hbm, onesv)
        pltpu.sync_copy(dst_hbm.at[pl.ds(sid * rows_per, rows_per)], dstv)
        pltpu.sync_copy(zeros_hbm, deg_sh.at[pl.ds(sid * RPW, RPW)])
        plsc.subcore_barrier()

        def body(j, carry):
            pltpu.sync_copy(onesv, deg_sh.at[dstv.at[j]], add=True)
            return carry

        lax.fori_loop(0, rows_per, body, 0)
        plsc.subcore_barrier()

        pltpu.sync_copy(deg_sh.at[pl.ds(sid * RPW, RPW)], degv)

        def newton(k, carry):
            x = degv[pl.ds(k * 16, 16)] + 1.0
            i = plsc.bitcast(x, jnp.int32)
            y = plsc.bitcast(jnp.int32(0x5F3759DF) - (i >> 1), jnp.float32)
            for _ in range(3):
                y = y * (1.5 - 0.5 * x * y * y)
            dinvv[pl.ds(k * 16, 16)] = y
            return carry

        lax.fori_loop(0, RPW // 16, newton, 0)
        pltpu.sync_copy(dinvv, dinv_hbm.at[pl.ds(sid * RPW, RPW)])


@functools.partial(
    pl.kernel,
    out_type=jax.ShapeDtypeStruct((NC, N_ACC, D), jnp.float32),
    mesh=_mesh,
    compiler_params=pltpu.CompilerParams(needs_layout_passes=False),
    scratch_types=[
        pltpu.VMEM((16, CH), jnp.int32),
        pltpu.VMEM((16, CH), jnp.int32),
        pltpu.VMEM((CH, D), jnp.float32),
        pltpu.VMEM((CH, D), jnp.float32),
        pltpu.VMEM_SHARED((N_ACC, D), jnp.float32),
        pltpu.SemaphoreType.DMA,
        pltpu.SemaphoreType.DMA,
    ],
)
def _segsum(g_hbm, src_hbm, dst_hbm, zeros_hbm, acc_hbm,
            srcv, dstv, rows_a, rows_b, acc_sh, sem_a, sem_b):
    cid = lax.axis_index("c")
    sid = lax.axis_index("s")
    wid = cid * NS + sid

    pltpu.sync_copy(zeros_hbm, acc_sh.at[pl.ds(sid * RPW, RPW)])
    plsc.subcore_barrier()

    def super_body(s, carry):
        pltpu.sync_copy(src_hbm.at[pl.ds((wid * NCH // 16 + s) * 16, 16)],
                        srcv)
        pltpu.sync_copy(dst_hbm.at[pl.ds((wid * NCH // 16 + s) * 16, 16)],
                        dstv)
        pltpu.async_copy(g_hbm.at[srcv.at[0]], rows_a, sem_a)

        def body(t, carry2):
            j = 2 * t
            pltpu.make_async_copy(g_hbm.at[srcv.at[j]], rows_a, sem_a).wait()
            pltpu.async_copy(g_hbm.at[srcv.at[j + 1]], rows_b, sem_b)
            pltpu.sync_copy(rows_a, acc_sh.at[dstv.at[j]], add=True)
            pltpu.make_async_copy(g_hbm.at[srcv.at[j + 1]], rows_b,
                                  sem_b).wait()

            @pl.when(t < 7)
            def _():
                pltpu.async_copy(g_hbm.at[srcv.at[j + 2]], rows_a, sem_a)

            pltpu.sync_copy(rows_b, acc_sh.at[dstv.at[j + 1]], add=True)
            return carry2

        lax.fori_loop(0, 8, body, 0)
        return carry

    lax.fori_loop(0, NCH // 16, super_body, 0)
    plsc.subcore_barrier()
    pltpu.sync_copy(acc_sh.at[pl.ds(sid * RPW, RPW)],
                    acc_hbm.at[cid, pl.ds(sid * RPW, RPW)])


def _scale_mm_body(x_ref, w_ref, dinv_ref, g_ref):
    g_ref[...] = jnp.dot(x_ref[...], w_ref[...],
                         preferred_element_type=jnp.float32) * dinv_ref[...]


def _mid_body(acc_ref, g_ref, dinv_ref, b_ref, w_ref, out_ref):
    o = (acc_ref[0] + acc_ref[1] + g_ref[...]) * dinv_ref[...] + b_ref[...]
    h = jnp.maximum(o, 0.0)
    out_ref[...] = jnp.dot(h, w_ref[...],
                           preferred_element_type=jnp.float32) * dinv_ref[...]


def _final_body(acc_ref, g_ref, dinv_ref, b_ref, out_ref):
    out_ref[...] = ((acc_ref[0] + acc_ref[1] + g_ref[...]) * dinv_ref[...]
                    + b_ref[...])


_BLK = 1000
_GRID = (N // _BLK,)
_row_spec = pl.BlockSpec((_BLK, D), lambda i: (i, 0))
_dinv_spec = pl.BlockSpec((_BLK, 1), lambda i: (i, 0))
_w_spec = pl.BlockSpec((D, D), lambda i: (0, 0))
_b_spec = pl.BlockSpec((1, D), lambda i: (0, 0))
_acc_spec = pl.BlockSpec((NC, _BLK, D), lambda i: (0, i, 0))
_row_out = jax.ShapeDtypeStruct((N, D), jnp.float32)

_scale_mm = pl.pallas_call(
    _scale_mm_body, grid=_GRID,
    in_specs=[_row_spec, _w_spec, _dinv_spec],
    out_specs=_row_spec, out_shape=_row_out)

_mid = pl.pallas_call(
    _mid_body, grid=_GRID,
    in_specs=[_acc_spec, _row_spec, _dinv_spec, _b_spec, _w_spec],
    out_specs=_row_spec, out_shape=_row_out)

_final = pl.pallas_call(
    _final_body, grid=_GRID,
    in_specs=[_acc_spec, _row_spec, _dinv_spec, _b_spec],
    out_specs=_row_spec, out_shape=_row_out)


def kernel(x, edge_index, W1, b1, W2, b2):
    src = edge_index[0].astype(jnp.int32)
    dst = edge_index[1].astype(jnp.int32)
    npad = E_PAD - E
    pi = jnp.arange(npad, dtype=jnp.int32)
    src2d = jnp.concatenate([src, pi % N]).reshape(NROW, CH)
    dst2d = jnp.concatenate([dst, N + pi % (N_ACC - N)]).reshape(NROW, CH)

    ones1 = jnp.ones((CH,), jnp.float32)
    zeros1 = jnp.zeros((RPW,), jnp.float32)
    zrows = jnp.zeros((RPW, D), jnp.float32)

    dinv_flat = _deg_dinv(dst2d, ones1, zeros1)
    dinv = dinv_flat[:N, None]

    g1 = _scale_mm(x, W1, dinv)
    acc1 = _segsum(g1, src2d, dst2d, zrows)
    g2 = _mid(acc1, g1, dinv, b1[None, :], W2)
    acc2 = _segsum(g2, src2d, dst2d, zrows)
    return _final(acc2, g2, dinv, b2[None, :])


# speedup vs baseline: 27.1754x; 1.0222x over previous
import functools

import jax
import jax.numpy as jnp
from jax import lax
from jax.experimental import pallas as pl
from jax.experimental.pallas import tpu as pltpu
from jax.experimental.pallas import tpu_sc as plsc

N = 10000
D = 128
E = 320000
CH = 64
NC = 2
NS = 16
NW = NC * NS
NCHK = 160
E_PAD = NW * NCHK * CH
NROW = E_PAD // CH
N_ACC = 10240
RPW = N_ACC // NS
OPW = N // NS

_mesh = plsc.VectorSubcoreMesh(core_axis_name="c", subcore_axis_name="s")


@functools.partial(
    pl.kernel,
    out_type=jax.ShapeDtypeStruct((N_ACC,), jnp.float32),
    mesh=_mesh,
    compiler_params=pltpu.CompilerParams(needs_layout_passes=False),
    scratch_types=[
        pltpu.VMEM((NROW // NS, CH), jnp.int32),
        pltpu.VMEM((CH,), jnp.float32),
        pltpu.VMEM((RPW,), jnp.float32),
        pltpu.VMEM((RPW,), jnp.float32),
        pltpu.VMEM_SHARED((N_ACC,), jnp.float32),
        pltpu.SemaphoreType.DMA,
    ],
)
def _deg_dinv(dst_hbm, ones_hbm, zeros_hbm, dinv_hbm,
              dstv, onesv, degv, dinvv, deg_sh, sem):
    cid = lax.axis_index("c")
    sid = lax.axis_index("s")
    rows_per = NROW // NS

    @pl.when(cid == 0)
    def _():
        pltpu.sync_copy(ones_hbm, onesv)
        pltpu.sync_copy(dst_hbm.at[pl.ds(sid * rows_per, rows_per)], dstv)
        pltpu.sync_copy(zeros_hbm, deg_sh.at[pl.ds(sid * RPW, RPW)])
        plsc.subcore_barrier()

        def body(j, carry):
            pltpu.sync_copy(onesv, deg_sh.at[dstv.at[j]], add=True)
            return carry

        lax.fori_loop(0, rows_per, body, 0)
        plsc.subcore_barrier()

        pltpu.sync_copy(deg_sh.at[pl.ds(sid * RPW, RPW)], degv)

        def newton(k, carry):
            x = degv[pl.ds(k * 16, 16)] + 1.0
            i = plsc.bitcast(x, jnp.int32)
            y = plsc.bitcast(jnp.int32(0x5F3759DF) - (i >> 1), jnp.float32)
            for _ in range(3):
                y = y * (1.5 - 0.5 * x * y * y)
            dinvv[pl.ds(k * 16, 16)] = y
            return carry

        lax.fori_loop(0, RPW // 16, newton, 0)
        pltpu.sync_copy(dinvv, dinv_hbm.at[pl.ds(sid * RPW, RPW)])


@functools.partial(
    pl.kernel,
    out_type=jax.ShapeDtypeStruct((NC, N_ACC, D), jnp.float32),
    mesh=_mesh,
    compiler_params=pltpu.CompilerParams(needs_layout_passes=False),
    scratch_types=[
        pltpu.VMEM((NCHK // 4, CH), jnp.int32),
        pltpu.VMEM((NCHK // 4, CH), jnp.int32),
        [pltpu.VMEM((CH, D), jnp.float32)] * 4,
        pltpu.VMEM_SHARED((N_ACC, D), jnp.float32),
        [pltpu.SemaphoreType.DMA] * 4,
        [pltpu.SemaphoreType.DMA] * 4,
    ],
)
def _segsum(g_hbm, src_hbm, dst_hbm, zeros_hbm, acc_hbm,
            srcv, dstv, rows, acc_sh, sg, ss):
    cid = lax.axis_index("c")
    sid = lax.axis_index("s")
    wid = cid * NS + sid
    qtr = NCHK // 4

    pltpu.sync_copy(zeros_hbm, acc_sh.at[pl.ds(sid * RPW, RPW)])
    plsc.subcore_barrier()

    for h in range(4):
        base = wid * NCHK + h * qtr
        pltpu.sync_copy(src_hbm.at[pl.ds(base, qtr)], srcv)
        pltpu.sync_copy(dst_hbm.at[pl.ds(base, qtr)], dstv)
        pltpu.async_copy(g_hbm.at[srcv.at[0]], rows[0], sg[0])
        pltpu.async_copy(g_hbm.at[srcv.at[1]], rows[1], sg[1])

        def t_body(t, carry):
            for b in range(4):
                j = 4 * t + b
                o = (b + 2) % 4
                pltpu.make_async_copy(g_hbm.at[srcv.at[j]], rows[b],
                                      sg[b]).wait()
                pltpu.async_copy(rows[b], acc_sh.at[dstv.at[j]], ss[b],
                                 add=True)

                @pl.when(j >= 2)
                def _():
                    pltpu.make_async_copy(rows[o], acc_sh.at[dstv.at[0]],
                                          ss[o]).wait()

                @pl.when(j + 2 < qtr)
                def _():
                    pltpu.async_copy(g_hbm.at[srcv.at[j + 2]], rows[o], sg[o])

            return carry

        lax.fori_loop(0, qtr // 4, t_body, 0)
        pltpu.make_async_copy(rows[2], acc_sh.at[dstv.at[0]], ss[2]).wait()
        pltpu.make_async_copy(rows[3], acc_sh.at[dstv.at[0]], ss[3]).wait()

    plsc.subcore_barrier()
    pltpu.sync_copy(acc_sh.at[pl.ds(sid * RPW, RPW)],
                    acc_hbm.at[cid, pl.ds(sid * RPW, RPW)])


def _scale_mm_body(x_ref, w_ref, dinv_ref, g_ref):
    g_ref[...] = jnp.dot(x_ref[...], w_ref[...],
                         preferred_element_type=jnp.float32) * dinv_ref[...]


def _mid_body(acc_ref, g_ref, dinv_ref, b_ref, w_ref, out_ref):
    o = (acc_ref[0] + acc_ref[1] + g_ref[...]) * dinv_ref[...] + b_ref[...]
    h = jnp.maximum(o, 0.0)
    out_ref[...] = jnp.dot(h, w_ref[...],
                           preferred_element_type=jnp.float32) * dinv_ref[...]


def _final_body(acc_ref, g_ref, dinv_ref, b_ref, out_ref):
    out_ref[...] = ((acc_ref[0] + acc_ref[1] + g_ref[...]) * dinv_ref[...]
                    + b_ref[...])


_BLK = 1000
_GRID = (N // _BLK,)
_row_spec = pl.BlockSpec((_BLK, D), lambda i: (i, 0))
_dinv_spec = pl.BlockSpec((_BLK, 1), lambda i: (i, 0))
_w_spec = pl.BlockSpec((D, D), lambda i: (0, 0))
_b_spec = pl.BlockSpec((1, D), lambda i: (0, 0))
_acc_spec = pl.BlockSpec((NC, _BLK, D), lambda i: (0, i, 0))
_row_out = jax.ShapeDtypeStruct((N, D), jnp.float32)

_scale_mm = pl.pallas_call(
    _scale_mm_body, grid=_GRID,
    in_specs=[_row_spec, _w_spec, _dinv_spec],
    out_specs=_row_spec, out_shape=_row_out)

_mid = pl.pallas_call(
    _mid_body, grid=_GRID,
    in_specs=[_acc_spec, _row_spec, _dinv_spec, _b_spec, _w_spec],
    out_specs=_row_spec, out_shape=_row_out)

_final = pl.pallas_call(
    _final_body, grid=_GRID,
    in_specs=[_acc_spec, _row_spec, _dinv_spec, _b_spec],
    out_specs=_row_spec, out_shape=_row_out)


def kernel(x, edge_index, W1, b1, W2, b2):
    src = edge_index[0].astype(jnp.int32)
    dst = edge_index[1].astype(jnp.int32)
    npad = E_PAD - E
    pi = jnp.arange(npad, dtype=jnp.int32)
    src2d = jnp.concatenate([src, pi % N]).reshape(NROW, CH)
    dst2d = jnp.concatenate([dst, N + pi % (N_ACC - N)]).reshape(NROW, CH)

    ones1 = jnp.ones((CH,), jnp.float32)
    zeros1 = jnp.zeros((RPW,), jnp.float32)
    zrows = jnp.zeros((RPW, D), jnp.float32)

    dinv_flat = _deg_dinv(dst2d, ones1, zeros1)
    dinv = dinv_flat[:N, None]

    g1 = _scale_mm(x, W1, dinv)
    acc1 = _segsum(g1, src2d, dst2d, zrows)
    g2 = _mid(acc1, g1, dinv, b1[None, :], W2)
    acc2 = _segsum(g2, src2d, dst2d, zrows)
    return _final(acc2, g2, dinv, b2[None, :])


# speedup vs baseline: 27.8126x; 1.0234x over previous
import functools

import jax
import jax.numpy as jnp
from jax import lax
from jax.experimental import pallas as pl
from jax.experimental.pallas import tpu as pltpu
from jax.experimental.pallas import tpu_sc as plsc

N = 10000
D = 128
E = 320000
CH = 64
NC = 2
NS = 16
NW = NC * NS
NCHK = 160
E_PAD = NW * NCHK * CH
NROW = E_PAD // CH
N_ACC = 10240
RPW = N_ACC // NS
OPW = N // NS

_mesh = plsc.VectorSubcoreMesh(core_axis_name="c", subcore_axis_name="s")


@functools.partial(
    pl.kernel,
    out_type=jax.ShapeDtypeStruct((NC, N_ACC), jnp.float32),
    mesh=_mesh,
    compiler_params=pltpu.CompilerParams(needs_layout_passes=False),
    scratch_types=[
        pltpu.VMEM((NROW // NW, CH), jnp.int32),
        pltpu.VMEM((CH,), jnp.float32),
        pltpu.VMEM_SHARED((N_ACC,), jnp.float32),
        pltpu.SemaphoreType.DMA,
    ],
)
def _deg(dst_hbm, ones_hbm, zeros_hbm, deg_hbm, dstv, onesv, deg_sh, sem):
    cid = lax.axis_index("c")
    sid = lax.axis_index("s")
    rows_per = NROW // NW

    pltpu.sync_copy(ones_hbm, onesv)
    pltpu.sync_copy(dst_hbm.at[pl.ds((cid * NS + sid) * rows_per, rows_per)],
                    dstv)
    pltpu.sync_copy(zeros_hbm, deg_sh.at[pl.ds(sid * RPW, RPW)])
    plsc.subcore_barrier()

    def body(j, carry):
        pltpu.sync_copy(onesv, deg_sh.at[dstv.at[j]], add=True)
        return carry

    lax.fori_loop(0, rows_per, body, 0)
    plsc.subcore_barrier()
    pltpu.sync_copy(deg_sh.at[pl.ds(sid * RPW, RPW)],
                    deg_hbm.at[cid, pl.ds(sid * RPW, RPW)])


@functools.partial(
    pl.kernel,
    out_type=jax.ShapeDtypeStruct((NC, N_ACC, D), jnp.float32),
    mesh=_mesh,
    compiler_params=pltpu.CompilerParams(needs_layout_passes=False),
    scratch_types=[
        pltpu.VMEM((NCHK // 4, CH), jnp.int32),
        pltpu.VMEM((NCHK // 4, CH), jnp.int32),
        [pltpu.VMEM((CH, D), jnp.float32)] * 4,
        pltpu.VMEM_SHARED((N_ACC, D), jnp.float32),
        [pltpu.SemaphoreType.DMA] * 4,
        [pltpu.SemaphoreType.DMA] * 4,
    ],
)
def _segsum(g_hbm, src_hbm, dst_hbm, zeros_hbm, acc_hbm,
            srcv, dstv, rows, acc_sh, sg, ss):
    cid = lax.axis_index("c")
    sid = lax.axis_index("s")
    wid = cid * NS + sid
    qtr = NCHK // 4

    pltpu.sync_copy(zeros_hbm, acc_sh.at[pl.ds(sid * RPW, RPW)])
    plsc.subcore_barrier()

    for h in range(4):
        base = wid * NCHK + h * qtr
        pltpu.sync_copy(src_hbm.at[pl.ds(base, qtr)], srcv)
        pltpu.sync_copy(dst_hbm.at[pl.ds(base, qtr)], dstv)
        pltpu.async_copy(g_hbm.at[srcv.at[0]], rows[0], sg[0])
        pltpu.async_copy(g_hbm.at[srcv.at[1]], rows[1], sg[1])

        def t_body(t, carry):
            for b in range(4):
                j = 4 * t + b
                o = (b + 2) % 4
                pltpu.make_async_copy(g_hbm.at[srcv.at[j]], rows[b],
                                      sg[b]).wait()
                pltpu.async_copy(rows[b], acc_sh.at[dstv.at[j]], ss[b],
                                 add=True)

                @pl.when(j >= 2)
                def _():
                    pltpu.make_async_copy(rows[o], acc_sh.at[dstv.at[0]],
                                          ss[o]).wait()

                @pl.when(j + 2 < qtr)
                def _():
                    pltpu.async_copy(g_hbm.at[srcv.at[j + 2]], rows[o], sg[o])

            return carry

        lax.fori_loop(0, qtr // 4, t_body, 0)
        pltpu.make_async_copy(rows[2], acc_sh.at[dstv.at[0]], ss[2]).wait()
        pltpu.make_async_copy(rows[3], acc_sh.at[dstv.at[0]], ss[3]).wait()

    plsc.subcore_barrier()
    pltpu.sync_copy(acc_sh.at[pl.ds(sid * RPW, RPW)],
                    acc_hbm.at[cid, pl.ds(sid * RPW, RPW)])


def _dinv(deg_ref):
    return lax.rsqrt(deg_ref[0] + deg_ref[1] + 1.0)


def _mm_body(x_ref, w_ref, h_ref):
    h_ref[...] = jnp.dot(x_ref[...], w_ref[...],
                         preferred_element_type=jnp.float32)


def _scale_body(h_ref, deg_ref, g_ref):
    g_ref[...] = h_ref[...] * _dinv(deg_ref)


def _mid_body(acc_ref, g_ref, deg_ref, b_ref, w_ref, out_ref):
    dinv = _dinv(deg_ref)
    o = (acc_ref[0] + acc_ref[1] + g_ref[...]) * dinv + b_ref[...]
    h = jnp.maximum(o, 0.0)
    out_ref[...] = jnp.dot(h, w_ref[...],
                           preferred_element_type=jnp.float32) * dinv


def _final_body(acc_ref, g_ref, deg_ref, b_ref, out_ref):
    out_ref[...] = ((acc_ref[0] + acc_ref[1] + g_ref[...]) * _dinv(deg_ref)
                    + b_ref[...])


_BLK = 1000
_GRID = (N // _BLK,)
_row_spec = pl.BlockSpec((_BLK, D), lambda i: (i, 0))
_deg_spec = pl.BlockSpec((NC, _BLK, 1), lambda i: (0, i, 0))
_w_spec = pl.BlockSpec((D, D), lambda i: (0, 0))
_b_spec = pl.BlockSpec((1, D), lambda i: (0, 0))
_acc_spec = pl.BlockSpec((NC, _BLK, D), lambda i: (0, i, 0))
_row_out = jax.ShapeDtypeStruct((N, D), jnp.float32)

_mm = pl.pallas_call(
    _mm_body, grid=_GRID,
    in_specs=[_row_spec, _w_spec],
    out_specs=_row_spec, out_shape=_row_out)

_scale = pl.pallas_call(
    _scale_body, grid=_GRID,
    in_specs=[_row_spec, _deg_spec],
    out_specs=_row_spec, out_shape=_row_out)

_mid = pl.pallas_call(
    _mid_body, grid=_GRID,
    in_specs=[_acc_spec, _row_spec, _deg_spec, _b_spec, _w_spec],
    out_specs=_row_spec, out_shape=_row_out)

_final = pl.pallas_call(
    _final_body, grid=_GRID,
    in_specs=[_acc_spec, _row_spec, _deg_spec, _b_spec],
    out_specs=_row_spec, out_shape=_row_out)


def kernel(x, edge_index, W1, b1, W2, b2):
    src = edge_index[0].astype(jnp.int32)
    dst = edge_index[1].astype(jnp.int32)
    npad = E_PAD - E
    pi = jnp.arange(npad, dtype=jnp.int32)
    src2d = jnp.concatenate([src, pi % N]).reshape(NROW, CH)
    dst2d = jnp.concatenate([dst, N + pi % (N_ACC - N)]).reshape(NROW, CH)

    ones1 = jnp.ones((CH,), jnp.float32)
    zeros1 = jnp.zeros((RPW,), jnp.float32)
    zrows = jnp.zeros((RPW, D), jnp.float32)

    degp = _deg(dst2d, ones1, zeros1)
    degc = degp[:, :N, None]
    h1 = _mm(x, W1)
    g1 = _scale(h1, degc)

    acc1 = _segsum(g1, src2d, dst2d, zrows)
    g2 = _mid(acc1, g1, degc, b1[None, :], W2)
    acc2 = _segsum(g2, src2d, dst2d, zrows)
    return _final(acc2, g2, degc, b2[None, :])


# speedup vs baseline: 28.2685x; 1.0164x over previous
import functools

import jax
import jax.numpy as jnp
from jax import lax
from jax.experimental import pallas as pl
from jax.experimental.pallas import tpu as pltpu
from jax.experimental.pallas import tpu_sc as plsc

N = 10000
D = 128
E = 320000
CH = 64
NC = 2
NS = 16
NW = NC * NS
NCHK = 160
E_PAD = NW * NCHK * CH
NROW = E_PAD // CH
N_ACC = 10240
RPW = N_ACC // NS
OPW = N // NS

_mesh = plsc.VectorSubcoreMesh(core_axis_name="c", subcore_axis_name="s")


@functools.partial(
    pl.kernel,
    out_type=jax.ShapeDtypeStruct((NC, N_ACC), jnp.float32),
    mesh=_mesh,
    compiler_params=pltpu.CompilerParams(needs_layout_passes=False),
    scratch_types=[
        pltpu.VMEM((E_PAD // 128 // NW, 128), jnp.int32),
        pltpu.VMEM((128,), jnp.float32),
        pltpu.VMEM_SHARED((N_ACC,), jnp.float32),
        pltpu.SemaphoreType.DMA,
    ],
)
def _deg(dst_hbm, ones_hbm, zeros_hbm, deg_hbm, dstv, onesv, deg_sh, sem):
    cid = lax.axis_index("c")
    sid = lax.axis_index("s")
    rows_per = E_PAD // 128 // NW

    pltpu.sync_copy(ones_hbm, onesv)
    pltpu.sync_copy(dst_hbm.at[pl.ds((cid * NS + sid) * rows_per, rows_per)],
                    dstv)
    pltpu.sync_copy(zeros_hbm, deg_sh.at[pl.ds(sid * RPW, RPW)])
    plsc.subcore_barrier()

    def body(j, carry):
        pltpu.async_copy(onesv, deg_sh.at[dstv.at[j]], sem, add=True)

        @pl.when(j >= 16)
        def _():
            pltpu.make_async_copy(onesv, deg_sh.at[dstv.at[0]], sem).wait()

        return carry

    lax.fori_loop(0, rows_per, body, 0)

    def drain(j, carry):
        pltpu.make_async_copy(onesv, deg_sh.at[dstv.at[0]], sem).wait()
        return carry

    lax.fori_loop(0, 16, drain, 0)
    plsc.subcore_barrier()
    pltpu.sync_copy(deg_sh.at[pl.ds(sid * RPW, RPW)],
                    deg_hbm.at[cid, pl.ds(sid * RPW, RPW)])


@functools.partial(
    pl.kernel,
    out_type=jax.ShapeDtypeStruct((NC, N_ACC, D), jnp.float32),
    mesh=_mesh,
    compiler_params=pltpu.CompilerParams(needs_layout_passes=False),
    scratch_types=[
        pltpu.VMEM((NCHK // 4, CH), jnp.int32),
        pltpu.VMEM((NCHK // 4, CH), jnp.int32),
        [pltpu.VMEM((CH, D), jnp.float32)] * 4,
        pltpu.VMEM_SHARED((N_ACC, D), jnp.float32),
        [pltpu.SemaphoreType.DMA] * 4,
        [pltpu.SemaphoreType.DMA] * 4,
    ],
)
def _segsum(g_hbm, src_hbm, dst_hbm, zeros_hbm, acc_hbm,
            srcv, dstv, rows, acc_sh, sg, ss):
    cid = lax.axis_index("c")
    sid = lax.axis_index("s")
    wid = cid * NS + sid
    qtr = NCHK // 4

    pltpu.sync_copy(zeros_hbm, acc_sh.at[pl.ds(sid * RPW, RPW)])
    plsc.subcore_barrier()

    for h in range(4):
        base = wid * NCHK + h * qtr
        pltpu.sync_copy(src_hbm.at[pl.ds(base, qtr)], srcv)
        pltpu.sync_copy(dst_hbm.at[pl.ds(base, qtr)], dstv)
        pltpu.async_copy(g_hbm.at[srcv.at[0]], rows[0], sg[0])
        pltpu.async_copy(g_hbm.at[srcv.at[1]], rows[1], sg[1])

        def t_body(t, carry):
            for b in range(4):
                j = 4 * t + b
                o = (b + 2) % 4
                pltpu.make_async_copy(g_hbm.at[srcv.at[j]], rows[b],
                                      sg[b]).wait()
                pltpu.async_copy(rows[b], acc_sh.at[dstv.at[j]], ss[b],
                                 add=True)

                @pl.when(j >= 2)
                def _():
                    pltpu.make_async_copy(rows[o], acc_sh.at[dstv.at[0]],
                                          ss[o]).wait()

                @pl.when(j + 2 < qtr)
                def _():
                    pltpu.async_copy(g_hbm.at[srcv.at[j + 2]], rows[o], sg[o])

            return carry

        lax.fori_loop(0, qtr // 4, t_body, 0)
        pltpu.make_async_copy(rows[2], acc_sh.at[dstv.at[0]], ss[2]).wait()
        pltpu.make_async_copy(rows[3], acc_sh.at[dstv.at[0]], ss[3]).wait()

    plsc.subcore_barrier()
    pltpu.sync_copy(acc_sh.at[pl.ds(sid * RPW, RPW)],
                    acc_hbm.at[cid, pl.ds(sid * RPW, RPW)])


def _dinv(deg_ref):
    return lax.rsqrt(deg_ref[0] + deg_ref[1] + 1.0)


def _mm_body(x_ref, w_ref, h_ref):
    h_ref[...] = jnp.dot(x_ref[...], w_ref[...],
                         preferred_element_type=jnp.float32)


def _scale_body(h_ref, deg_ref, g_ref):
    g_ref[...] = h_ref[...] * _dinv(deg_ref)


def _mid_body(acc_ref, g_ref, deg_ref, b_ref, w_ref, out_ref):
    dinv = _dinv(deg_ref)
    o = (acc_ref[0] + acc_ref[1] + g_ref[...]) * dinv + b_ref[...]
    h = jnp.maximum(o, 0.0)
    out_ref[...] = jnp.dot(h, w_ref[...],
                           preferred_element_type=jnp.float32) * dinv


def _final_body(acc_ref, g_ref, deg_ref, b_ref, out_ref):
    out_ref[...] = ((acc_ref[0] + acc_ref[1] + g_ref[...]) * _dinv(deg_ref)
                    + b_ref[...])


_BLK = 1000
_GRID = (N // _BLK,)
_row_spec = pl.BlockSpec((_BLK, D), lambda i: (i, 0))
_deg_spec = pl.BlockSpec((NC, _BLK, 1), lambda i: (0, i, 0))
_w_spec = pl.BlockSpec((D, D), lambda i: (0, 0))
_b_spec = pl.BlockSpec((1, D), lambda i: (0, 0))
_acc_spec = pl.BlockSpec((NC, _BLK, D), lambda i: (0, i, 0))
_row_out = jax.ShapeDtypeStruct((N, D), jnp.float32)

_mm = pl.pallas_call(
    _mm_body, grid=_GRID,
    in_specs=[_row_spec, _w_spec],
    out_specs=_row_spec, out_shape=_row_out)

_scale = pl.pallas_call(
    _scale_body, grid=_GRID,
    in_specs=[_row_spec, _deg_spec],
    out_specs=_row_spec, out_shape=_row_out)

_mid = pl.pallas_call(
    _mid_body, grid=_GRID,
    in_specs=[_acc_spec, _row_spec, _deg_spec, _b_spec, _w_spec],
    out_specs=_row_spec, out_shape=_row_out)

_final = pl.pallas_call(
    _final_body, grid=_GRID,
    in_specs=[_acc_spec, _row_spec, _deg_spec, _b_spec],
    out_specs=_row_spec, out_shape=_row_out)


def kernel(x, edge_index, W1, b1, W2, b2):
    src = edge_index[0].astype(jnp.int32)
    dst = edge_index[1].astype(jnp.int32)
    npad = E_PAD - E
    pi = jnp.arange(npad, dtype=jnp.int32)
    src2d = jnp.concatenate([src, pi % N]).reshape(NROW, CH)
    dst_p = jnp.concatenate([dst, N + pi % (N_ACC - N)])
    dst2d = dst_p.reshape(NROW, CH)

    ones1 = jnp.ones((128,), jnp.float32)
    zeros1 = jnp.zeros((RPW,), jnp.float32)
    zrows = jnp.zeros((RPW, D), jnp.float32)

    degp = _deg(dst_p.reshape(E_PAD // 128, 128), ones1, zeros1)
    degc = degp[:, :N, None]
    h1 = _mm(x, W1)
    g1 = _scale(h1, degc)

    acc1 = _segsum(g1, src2d, dst2d, zrows)
    g2 = _mid(acc1, g1, degc, b1[None, :], W2)
    acc2 = _segsum(g2, src2d, dst2d, zrows)
    return _final(acc2, g2, degc, b2[None, :])


# speedup vs baseline: 28.8425x; 1.0203x over previous
import functools

import jax
import jax.numpy as jnp
from jax import lax
from jax.experimental import pallas as pl
from jax.experimental.pallas import tpu as pltpu
from jax.experimental.pallas import tpu_sc as plsc

N = 10000
D = 128
E = 320000
CH = 64
NC = 2
NS = 16
NW = NC * NS
NCHK = 160
E_PAD = NW * NCHK * CH
NROW = E_PAD // CH
N_ACC = 10240
RPW = N_ACC // NS
OPW = N // NS

_mesh = plsc.VectorSubcoreMesh(core_axis_name="c", subcore_axis_name="s")


@functools.partial(
    pl.kernel,
    out_type=jax.ShapeDtypeStruct((NC, N_ACC), jnp.float32),
    mesh=_mesh,
    compiler_params=pltpu.CompilerParams(needs_layout_passes=False),
    scratch_types=[
        pltpu.VMEM((E_PAD // 128 // NW, 128), jnp.int32),
        pltpu.VMEM((128,), jnp.float32),
        pltpu.VMEM_SHARED((N_ACC,), jnp.float32),
        pltpu.SemaphoreType.DMA,
    ],
)
def _deg(dst_hbm, ones_hbm, zeros_hbm, deg_hbm, dstv, onesv, deg_sh, sem):
    cid = lax.axis_index("c")
    sid = lax.axis_index("s")
    rows_per = E_PAD // 128 // NW

    pltpu.sync_copy(ones_hbm, onesv)
    pltpu.sync_copy(dst_hbm.at[pl.ds((cid * NS + sid) * rows_per, rows_per)],
                    dstv)
    pltpu.sync_copy(zeros_hbm, deg_sh.at[pl.ds(sid * RPW, RPW)])
    plsc.subcore_barrier()

    def body(j, carry):
        pltpu.async_copy(onesv, deg_sh.at[dstv.at[j]], sem, add=True)

        @pl.when(j >= 16)
        def _():
            pltpu.make_async_copy(onesv, deg_sh.at[dstv.at[0]], sem).wait()

        return carry

    lax.fori_loop(0, rows_per, body, 0)

    def drain(j, carry):
        pltpu.make_async_copy(onesv, deg_sh.at[dstv.at[0]], sem).wait()
        return carry

    lax.fori_loop(0, 16, drain, 0)
    plsc.subcore_barrier()
    pltpu.sync_copy(deg_sh.at[pl.ds(sid * RPW, RPW)],
                    deg_hbm.at[cid, pl.ds(sid * RPW, RPW)])


@functools.partial(
    pl.kernel,
    out_type=jax.ShapeDtypeStruct((NC, N_ACC, D), jnp.float32),
    mesh=_mesh,
    compiler_params=pltpu.CompilerParams(needs_layout_passes=False),
    scratch_types=[
        pltpu.VMEM((NCHK // 4, CH), jnp.int32),
        pltpu.VMEM((NCHK // 4, CH), jnp.int32),
        [pltpu.VMEM((CH, D), jnp.float32)] * 4,
        pltpu.VMEM_SHARED((N_ACC, D), jnp.float32),
        [pltpu.SemaphoreType.DMA] * 4,
        [pltpu.SemaphoreType.DMA] * 4,
    ],
)
def _segsum(g_hbm, src_hbm, dst_hbm, zeros_hbm, acc_hbm,
            srcv, dstv, rows, acc_sh, sg, ss):
    cid = lax.axis_index("c")
    sid = lax.axis_index("s")
    wid = cid * NS + sid
    qtr = NCHK // 4

    for h in range(4):
        base = wid * NCHK + h * qtr
        pltpu.sync_copy(src_hbm.at[pl.ds(base, qtr)], srcv)
        pltpu.sync_copy(dst_hbm.at[pl.ds(base, qtr)], dstv)
        pltpu.async_copy(g_hbm.at[srcv.at[0]], rows[0], sg[0])
        pltpu.async_copy(g_hbm.at[srcv.at[1]], rows[1], sg[1])
        if h == 0:
            pltpu.sync_copy(zeros_hbm, acc_sh.at[pl.ds(sid * RPW, RPW)])
            plsc.subcore_barrier()

        def t_body(t, carry):
            for b in range(4):
                j = 4 * t + b
                o = (b + 2) % 4
                pltpu.make_async_copy(g_hbm.at[srcv.at[j]], rows[b],
                                      sg[b]).wait()
                pltpu.async_copy(rows[b], acc_sh.at[dstv.at[j]], ss[b],
                                 add=True)

                @pl.when(j >= 2)
                def _():
                    pltpu.make_async_copy(rows[o], acc_sh.at[dstv.at[0]],
                                          ss[o]).wait()

                @pl.when(j + 2 < qtr)
                def _():
                    pltpu.async_copy(g_hbm.at[srcv.at[j + 2]], rows[o], sg[o])

            return carry

        lax.fori_loop(0, qtr // 4, t_body, 0)
        pltpu.make_async_copy(rows[2], acc_sh.at[dstv.at[0]], ss[2]).wait()
        pltpu.make_async_copy(rows[3], acc_sh.at[dstv.at[0]], ss[3]).wait()

    plsc.subcore_barrier()
    pltpu.sync_copy(acc_sh.at[pl.ds(sid * RPW, RPW)],
                    acc_hbm.at[cid, pl.ds(sid * RPW, RPW)])


def _dinv(deg_ref):
    return lax.rsqrt(deg_ref[0] + deg_ref[1] + 1.0)


def _scale_mm_body(x_ref, w_ref, deg_ref, g_ref):
    g_ref[...] = jnp.dot(x_ref[...], w_ref[...],
                         preferred_element_type=jnp.float32) * _dinv(deg_ref)


def _mid_body(acc_ref, g_ref, deg_ref, b_ref, w_ref, out_ref):
    dinv = _dinv(deg_ref)
    o = (acc_ref[0] + acc_ref[1] + g_ref[...]) * dinv + b_ref[...]
    h = jnp.maximum(o, 0.0)
    out_ref[...] = jnp.dot(h, w_ref[...],
                           preferred_element_type=jnp.float32) * dinv


def _final_body(acc_ref, g_ref, deg_ref, b_ref, out_ref):
    out_ref[...] = ((acc_ref[0] + acc_ref[1] + g_ref[...]) * _dinv(deg_ref)
                    + b_ref[...])


_BLK = 1000
_GRID = (N // _BLK,)
_row_spec = pl.BlockSpec((_BLK, D), lambda i: (i, 0))
_deg_spec = pl.BlockSpec((NC, _BLK, 1), lambda i: (0, i, 0))
_w_spec = pl.BlockSpec((D, D), lambda i: (0, 0))
_b_spec = pl.BlockSpec((1, D), lambda i: (0, 0))
_acc_spec = pl.BlockSpec((NC, _BLK, D), lambda i: (0, i, 0))
_row_out = jax.ShapeDtypeStruct((N, D), jnp.float32)

_scale_mm = pl.pallas_call(
    _scale_mm_body, grid=_GRID,
    in_specs=[_row_spec, _w_spec, _deg_spec],
    out_specs=_row_spec, out_shape=_row_out)

_mid = pl.pallas_call(
    _mid_body, grid=_GRID,
    in_specs=[_acc_spec, _row_spec, _deg_spec, _b_spec, _w_spec],
    out_specs=_row_spec, out_shape=_row_out)

_final = pl.pallas_call(
    _final_body, grid=_GRID,
    in_specs=[_acc_spec, _row_spec, _deg_spec, _b_spec],
    out_specs=_row_spec, out_shape=_row_out)


def kernel(x, edge_index, W1, b1, W2, b2):
    src = edge_index[0].astype(jnp.int32)
    dst = edge_index[1].astype(jnp.int32)
    npad = E_PAD - E
    pi = jnp.arange(npad, dtype=jnp.int32)
    src2d = jnp.concatenate([src, pi % N]).reshape(NROW, CH)
    dst_p = jnp.concatenate([dst, N + pi % (N_ACC - N)])
    dst2d = dst_p.reshape(NROW, CH)

    ones1 = jnp.ones((128,), jnp.float32)
    zeros1 = jnp.zeros((RPW,), jnp.float32)
    zrows = jnp.zeros((RPW, D), jnp.float32)

    degp = _deg(dst_p.reshape(E_PAD // 128, 128), ones1, zeros1)
    degc = degp[:, :N, None]
    g1 = _scale_mm(x, W1, degc)

    acc1 = _segsum(g1, src2d, dst2d, zrows)
    g2 = _mid(acc1, g1, degc, b1[None, :], W2)
    acc2 = _segsum(g2, src2d, dst2d, zrows)
    return _final(acc2, g2, degc, b2[None, :])


# speedup vs baseline: 29.5075x; 1.0231x over previous
import functools

import jax
import jax.numpy as jnp
from jax import lax
from jax.experimental import pallas as pl
from jax.experimental.pallas import tpu as pltpu
from jax.experimental.pallas import tpu_sc as plsc

N = 10000
D = 128
E = 320000
CH = 64
NC = 2
NS = 16
NW = NC * NS
NCHK = 160
E_PAD = NW * NCHK * CH
NROW = E_PAD // CH
N_ACC = 10240
RPW = N_ACC // NS
OPW = N // NS

_mesh = plsc.VectorSubcoreMesh(core_axis_name="c", subcore_axis_name="s")


@functools.partial(
    pl.kernel,
    out_type=jax.ShapeDtypeStruct((NC, N_ACC), jnp.float32),
    mesh=_mesh,
    compiler_params=pltpu.CompilerParams(needs_layout_passes=False),
    scratch_types=[
        pltpu.VMEM((E_PAD // 128 // NW, 128), jnp.int32),
        pltpu.VMEM((128,), jnp.float32),
        pltpu.VMEM_SHARED((N_ACC,), jnp.float32),
        pltpu.SemaphoreType.DMA,
    ],
)
def _deg(dst_hbm, ones_hbm, zeros_hbm, deg_hbm, dstv, onesv, deg_sh, sem):
    cid = lax.axis_index("c")
    sid = lax.axis_index("s")
    rows_per = E_PAD // 128 // NW

    pltpu.sync_copy(ones_hbm, onesv)
    pltpu.sync_copy(dst_hbm.at[pl.ds((cid * NS + sid) * rows_per, rows_per)],
                    dstv)
    pltpu.sync_copy(zeros_hbm, deg_sh.at[pl.ds(sid * RPW, RPW)])
    plsc.subcore_barrier()

    def body(j, carry):
        pltpu.async_copy(onesv, deg_sh.at[dstv.at[j]], sem, add=True)

        @pl.when(j >= 16)
        def _():
            pltpu.make_async_copy(onesv, deg_sh.at[dstv.at[0]], sem).wait()

        return carry

    lax.fori_loop(0, rows_per, body, 0)

    def drain(j, carry):
        pltpu.make_async_copy(onesv, deg_sh.at[dstv.at[0]], sem).wait()
        return carry

    lax.fori_loop(0, 16, drain, 0)
    plsc.subcore_barrier()
    pltpu.sync_copy(deg_sh.at[pl.ds(sid * RPW, RPW)],
                    deg_hbm.at[cid, pl.ds(sid * RPW, RPW)])


@functools.partial(
    pl.kernel,
    out_type=jax.ShapeDtypeStruct((NC, N_ACC, D), jnp.float32),
    mesh=_mesh,
    compiler_params=pltpu.CompilerParams(needs_layout_passes=False),
    scratch_types=[
        pltpu.VMEM((NCHK // 4, CH), jnp.int32),
        pltpu.VMEM((NCHK // 4, CH), jnp.int32),
        [pltpu.VMEM((CH, D), jnp.float32)] * 4,
        pltpu.VMEM_SHARED((N_ACC, D), jnp.float32),
        [pltpu.SemaphoreType.DMA] * 4,
        [pltpu.SemaphoreType.DMA] * 4,
    ],
)
def _segsum(g_hbm, src_hbm, dst_hbm, zeros_hbm, acc_hbm,
            srcv, dstv, rows, acc_sh, sg, ss):
    cid = lax.axis_index("c")
    sid = lax.axis_index("s")
    wid = cid * NS + sid
    qtr = NCHK // 4

    for h in range(4):
        base = wid * NCHK + h * qtr
        pltpu.sync_copy(src_hbm.at[pl.ds(base, qtr)], srcv)
        pltpu.sync_copy(dst_hbm.at[pl.ds(base, qtr)], dstv)
        pltpu.async_copy(g_hbm.at[srcv.at[0]], rows[0], sg[0])
        pltpu.async_copy(g_hbm.at[srcv.at[1]], rows[1], sg[1])
        if h == 0:
            pltpu.sync_copy(zeros_hbm, acc_sh.at[pl.ds(sid * RPW, RPW)])
            plsc.subcore_barrier()

        def t_body(t, carry):
            for b in range(4):
                j = 4 * t + b
                o = (b + 2) % 4
                pltpu.make_async_copy(g_hbm.at[srcv.at[j]], rows[b],
                                      sg[b]).wait()
                pltpu.async_copy(rows[b], acc_sh.at[dstv.at[j]], ss[b],
                                 add=True)

                @pl.when(j >= 2)
                def _():
                    pltpu.make_async_copy(rows[o], acc_sh.at[dstv.at[0]],
                                          ss[o]).wait()

                @pl.when(j + 2 < qtr)
                def _():
                    pltpu.async_copy(g_hbm.at[srcv.at[j + 2]], rows[o], sg[o])

            return carry

        lax.fori_loop(0, qtr // 4, t_body, 0)
        pltpu.make_async_copy(rows[2], acc_sh.at[dstv.at[0]], ss[2]).wait()
        pltpu.make_async_copy(rows[3], acc_sh.at[dstv.at[0]], ss[3]).wait()

    plsc.subcore_barrier()
    pltpu.sync_copy(acc_sh.at[pl.ds(sid * RPW, RPW)],
                    acc_hbm.at[cid, pl.ds(sid * RPW, RPW)])


def _dinv(deg_ref):
    return lax.rsqrt(deg_ref[0] + deg_ref[1] + 1.0)


def _scale_mm_body(x_ref, w_ref, deg_ref, g_ref):
    g_ref[...] = jnp.dot(x_ref[...], w_ref[...],
                         preferred_element_type=jnp.float32) * _dinv(deg_ref)


def _mid_body(acc_ref, g_ref, deg_ref, b_ref, w_ref, out_ref):
    dinv = _dinv(deg_ref)
    o = (acc_ref[0] + acc_ref[1] + g_ref[...]) * dinv + b_ref[...]
    h = jnp.maximum(o, 0.0)
    out_ref[...] = jnp.dot(h, w_ref[...],
                           preferred_element_type=jnp.float32) * dinv


def _final_body(acc_ref, g_ref, deg_ref, b_ref, out_ref):
    out_ref[...] = ((acc_ref[0] + acc_ref[1] + g_ref[...]) * _dinv(deg_ref)
                    + b_ref[...])


_BLK = 2000
_GRID = (N // _BLK,)
_row_spec = pl.BlockSpec((_BLK, D), lambda i: (i, 0))
_deg_spec = pl.BlockSpec((NC, _BLK, 1), lambda i: (0, i, 0))
_w_spec = pl.BlockSpec((D, D), lambda i: (0, 0))
_b_spec = pl.BlockSpec((1, D), lambda i: (0, 0))
_acc_spec = pl.BlockSpec((NC, _BLK, D), lambda i: (0, i, 0))
_row_out = jax.ShapeDtypeStruct((N, D), jnp.float32)

_scale_mm = pl.pallas_call(
    _scale_mm_body, grid=_GRID,
    in_specs=[_row_spec, _w_spec, _deg_spec],
    out_specs=_row_spec, out_shape=_row_out)

_mid = pl.pallas_call(
    _mid_body, grid=_GRID,
    in_specs=[_acc_spec, _row_spec, _deg_spec, _b_spec, _w_spec],
    out_specs=_row_spec, out_shape=_row_out)

_final = pl.pallas_call(
    _final_body, grid=_GRID,
    in_specs=[_acc_spec, _row_spec, _deg_spec, _b_spec],
    out_specs=_row_spec, out_shape=_row_out)


def kernel(x, edge_index, W1, b1, W2, b2):
    src = edge_index[0].astype(jnp.int32)
    dst = edge_index[1].astype(jnp.int32)
    npad = E_PAD - E
    pi = jnp.arange(npad, dtype=jnp.int32)
    src2d = jnp.concatenate([src, pi % N]).reshape(NROW, CH)
    dst_p = jnp.concatenate([dst, N + pi % (N_ACC - N)])
    dst2d = dst_p.reshape(NROW, CH)

    ones1 = jnp.ones((128,), jnp.float32)
    zeros1 = jnp.zeros((RPW,), jnp.float32)
    zrows = jnp.zeros((RPW, D), jnp.float32)

    degp = _deg(dst_p.reshape(E_PAD // 128, 128), ones1, zeros1)
    degc = degp[:, :N, None]
    g1 = _scale_mm(x, W1, degc)

    acc1 = _segsum(g1, src2d, dst2d, zrows)
    g2 = _mid(acc1, g1, degc, b1[None, :], W2)
    acc2 = _segsum(g2, src2d, dst2d, zrows)
    return _final(acc2, g2, degc, b2[None, :])
